# baseline (speedup 1.0000x reference)
import jax
import jax.numpy as jnp
from jax.experimental import pallas as pl


def _copy_body(x_ref, o_ref):
    o_ref[...] = x_ref[...]


def kernel(x, sources, targets, batch, counts, total, emb, conv_w, res_w,
           fc1_w, fc1_b, fc2_w, fc2_b, fc3_w, fc3_b):
    n = x.shape[0]
    w = jnp.ones((sources.shape[0],), jnp.float32)
    deg_in = jnp.ones((n,), jnp.float32).at[targets].add(w)
    deg_out = jnp.ones((n,), jnp.float32).at[sources].add(w)
    norm = (1.0 / deg_in)[:, None]
    norm_t = (1.0 / deg_out)[:, None]

    def conv(h, src, tgt, nrm, W):
        agg = h.at[tgt].add(h[src])
        return (nrm * agg) @ W

    def biconv(h, W2):
        out = conv(h, sources, targets, norm, W2[0])
        back = conv(h, targets, sources, norm_t, W2[1])
        return out + back

    def bn(h):
        m = jnp.mean(h, axis=0)
        v = jnp.var(h, axis=0)
        return (h - m) / jnp.sqrt(v + 1e-5)

    h = emb[x]
    h = biconv(h, conv_w)
    for i in range(4):
        save = h
        h = bn(jax.nn.relu(h))
        h = biconv(h, res_w[i, 0])
        h = bn(jax.nn.relu(h))
        h = biconv(h, res_w[i, 1])
        h = save + h
    pooled = (jnp.zeros((counts.shape[0], h.shape[1]), jnp.float32)
              + 0.0 * total).at[batch].add(h)
    g = pooled / counts[:, None]
    g = jax.nn.relu(g @ fc1_w + fc1_b)
    g = jax.nn.relu(g @ fc2_w + fc2_b)
    g = g @ fc3_w + fc3_b
    g = jnp.squeeze(g, axis=-1)
    return pl.pallas_call(
        _copy_body,
        out_shape=jax.ShapeDtypeStruct(g.shape, g.dtype),
    )(g)


# speedup vs baseline: 4.8133x; 4.8133x over previous
import functools

import jax
import jax.numpy as jnp
from jax import lax
from jax.experimental import pallas as pl
from jax.experimental.pallas import tpu as pltpu
from jax.experimental.pallas import tpu_sc as plsc

F32 = jnp.float32
_N = 50000
_E = 800000
_G = 128
_C = 64
_HC = 32
_NP = 50176
_BLK = 1024
_NB = _NP // _BLK
_NTILE = 16
_RPT = _NP // _NTILE
_EPT = _E // _NTILE
_EK = 2000
_ECH = _EPT // _EK
_EKA = 400
_ECHA = _EPT // _EKA
_KP = _RPT // 2
_EPS = 1e-5
_PREC = lax.Precision.HIGHEST

@functools.cache
def _sc_mesh():
    return plsc.VectorSubcoreMesh(core_axis_name="c", subcore_axis_name="s",
                                  num_cores=2, num_subcores=_NTILE)


def _sc_agg_body(r2, srcf2, tgtf, srcb2, tgtb, aggf, aggb,
                 idx_g, idx_s, rows, acc):
    c = lax.axis_index("c")
    s = lax.axis_index("s")
    rowh = c * _NP + s * _RPT

    def direction(gsrc, tdst, out):
        pltpu.sync_copy(r2.at[pl.ds(rowh, _RPT)], acc.at[pl.ds(s * _RPT, _RPT)])
        plsc.subcore_barrier()

        def chunk(k, carry):
            off_g = c * _E + s * _EPT + k * _EKA
            off_t = s * _EPT + k * _EKA
            pltpu.sync_copy(gsrc.at[pl.ds(off_g, _EKA)], idx_g)
            pltpu.sync_copy(tdst.at[pl.ds(off_t, _EKA)], idx_s)
            pltpu.sync_copy(r2.at[idx_g], rows)
            pltpu.sync_copy(rows, acc.at[idx_s], add=True)
            return carry

        lax.fori_loop(0, _ECHA, chunk, 0)
        plsc.subcore_barrier()
        pltpu.sync_copy(acc.at[pl.ds(s * _RPT, _RPT)], out.at[pl.ds(rowh, _RPT)])
        plsc.subcore_barrier()

    direction(srcf2, tgtf, aggf)
    direction(srcb2, tgtb, aggb)


@functools.cache
def _sc_agg():
    return pl.kernel(
        _sc_agg_body,
        out_type=[jax.ShapeDtypeStruct((2 * _NP, _HC), F32),
                  jax.ShapeDtypeStruct((2 * _NP, _HC), F32)],
        mesh=_sc_mesh(),
        compiler_params=pltpu.CompilerParams(use_tc_tiling_on_sc=False),
        scratch_types=[
            pltpu.VMEM((_EKA,), jnp.int32),
            pltpu.VMEM((_EKA,), jnp.int32),
            pltpu.VMEM((_EKA, _HC), F32),
            pltpu.VMEM_SHARED((_NP, _HC), F32),
        ],
    )


def _sc_deg_body(eidx2, degflat, idxbuf, ones, acc):
    c = lax.axis_index("c")
    s = lax.axis_index("s")

    def fill(i, carry):
        ones[i, :] = jnp.full((16,), 1.0, F32)
        return carry

    lax.fori_loop(0, _EK, fill, 0)
    pltpu.sync_copy(ones.at[pl.ds(0, _EK)], acc.at[pl.ds(s * _RPT, _EK)])
    pltpu.sync_copy(ones.at[pl.ds(0, _RPT - _EK)],
                    acc.at[pl.ds(s * _RPT + _EK, _RPT - _EK)])
    plsc.subcore_barrier()

    def chunk(k, carry):
        off = c * _E + s * _EPT + k * _EK
        pltpu.sync_copy(eidx2.at[pl.ds(off, _EK)], idxbuf)
        pltpu.sync_copy(ones, acc.at[idxbuf], add=True)
        return carry

    lax.fori_loop(0, _ECH, chunk, 0)
    plsc.subcore_barrier()
    pltpu.sync_copy(acc.at[pl.ds(s * _RPT, _RPT)],
                    degflat.at[pl.ds(c * _NP + s * _RPT, _RPT)])


@functools.cache
def _sc_deg():
    return pl.kernel(
        _sc_deg_body,
        out_type=jax.ShapeDtypeStruct((2 * _NP, 16), F32),
        mesh=_sc_mesh(),
        compiler_params=pltpu.CompilerParams(use_tc_tiling_on_sc=False),
        scratch_types=[
            pltpu.VMEM((_EK,), jnp.int32),
            pltpu.VMEM((_EK, 16), F32),
            pltpu.VMEM_SHARED((_NP, 16), F32),
        ],
    )


def _sc_pool_body(h2, batchp, pooledflat, idxbuf, rows, acc):
    c = lax.axis_index("c")
    s = lax.axis_index("s")

    @pl.when(s == 0)
    def _init():
        def fill(i, carry):
            rows[i, 0:16] = jnp.zeros((16,), F32)
            rows[i, 16:32] = jnp.zeros((16,), F32)
            return carry

        lax.fori_loop(0, _G, fill, 0)
        pltpu.sync_copy(rows.at[pl.ds(0, _G)], acc)

    plsc.subcore_barrier()

    def chunk(k, carry):
        roff = s * _RPT + k * _KP
        pltpu.sync_copy(h2.at[pl.ds(c * _NP + roff, _KP)], rows)
        pltpu.sync_copy(batchp.at[pl.ds(roff, _KP)], idxbuf)
        pltpu.sync_copy(rows, acc.at[idxbuf], add=True)
        return carry

    lax.fori_loop(0, 2, chunk, 0)
    plsc.subcore_barrier()

    @pl.when(s == 0)
    def _out():
        pltpu.sync_copy(acc, pooledflat.at[pl.ds(c * _G, _G)])


@functools.cache
def _sc_pool():
    return pl.kernel(
        _sc_pool_body,
        out_type=jax.ShapeDtypeStruct((2 * _G, _HC), F32),
        mesh=_sc_mesh(),
        compiler_params=pltpu.CompilerParams(use_tc_tiling_on_sc=False),
        scratch_types=[
            pltpu.VMEM((_KP,), jnp.int32),
            pltpu.VMEM((_KP, _HC), F32),
            pltpu.VMEM_SHARED((_G, _HC), F32),
        ],
    )


def _tc_embed_body(xf, emb, h_lo, h_hi):
    i = pl.program_id(0)
    xv = xf[...]
    e = emb[...]
    h = jnp.zeros((_BLK, _C), F32)
    for k in range(7):
        h = h + jnp.where(xv == float(k), 1.0, 0.0) * e[k][None, :]
    rowid = i * _BLK + lax.broadcasted_iota(jnp.int32, (_BLK, 1), 0)
    h = jnp.where(rowid < _N, h, 0.0)
    h_lo[...] = h[:, :_HC]
    h_hi[...] = h[:, _HC:]


def _tc_embed(xf, emb):
    return pl.pallas_call(
        _tc_embed_body,
        grid=(_NB,),
        in_specs=[
            pl.BlockSpec((_BLK, 1), lambda i: (i, 0)),
            pl.BlockSpec((7, _C), lambda i: (0, 0)),
        ],
        out_specs=[
            pl.BlockSpec((_BLK, _HC), lambda i: (i, 0)),
            pl.BlockSpec((_BLK, _HC), lambda i: (i, 0)),
        ],
        out_shape=[jax.ShapeDtypeStruct((_NP, _HC), F32),
                   jax.ShapeDtypeStruct((_NP, _HC), F32)],
    )(xf, emb)


def _tc_stage(aggf2, aggb2, degi, dego, w0, w1, sums, save,
              first, resid, last):
    def body(*refs):
        it = iter(refs)
        aggf_lo, aggf_hi, aggb_lo, aggb_hi, degi_r, dego_r, w0_r, w1_r = (
            next(it) for _ in range(8))
        sum_r = sumsq_r = save_r = None
        if not first:
            sum_r, sumsq_r = next(it), next(it)
        if resid:
            save_r = next(it)
        if last:
            h_lo, h_hi = next(it), next(it)
        else:
            r_lo, r_hi, sum_o, sumsq_o = (next(it) for _ in range(4))
            save_o = next(it) if (first or resid) else None

        i = pl.program_id(0)
        aggf = jnp.concatenate([aggf_lo[...], aggf_hi[...]], axis=1)
        aggb = jnp.concatenate([aggb_lo[...], aggb_hi[...]], axis=1)
        norm = 1.0 / degi_r[:, 0:1]
        normt = 1.0 / dego_r[:, 0:1]
        w0v, w1v = w0_r[...], w1_r[...]
        if not first:
            m = jnp.sum(sum_r[...], axis=(0, 1)) * (1.0 / _N)
            var = jnp.sum(sumsq_r[...], axis=(0, 1)) * (1.0 / _N) - m * m
            inv = lax.rsqrt(var + _EPS)
            aggf = aggf * inv[None, :]
            aggb = aggb * inv[None, :]
        h = (jnp.dot(norm * aggf, w0v, precision=_PREC) +
             jnp.dot(normt * aggb, w1v, precision=_PREC))
        if not first:
            bias = jnp.dot((m * inv)[None, :], w0v + w1v, precision=_PREC)
            h = h - bias
        if resid:
            h = h + save_r[...]
        rowid = i * _BLK + lax.broadcasted_iota(jnp.int32, (_BLK, 1), 0)
        h = jnp.where(rowid < _N, h, 0.0)
        if last:
            h_lo[...] = h[:, :_HC]
            h_hi[...] = h[:, _HC:]
        else:
            if save_o is not None:
                save_o[...] = h
            r = jnp.maximum(h, 0.0)
            r_lo[...] = r[:, :_HC]
            r_hi[...] = r[:, _HC:]
            sum_o[...] = jnp.sum(r, axis=0, keepdims=True)[None]
            sumsq_o[...] = jnp.sum(r * r, axis=0, keepdims=True)[None]

    half = pl.BlockSpec((_BLK, _HC), lambda i: (i, 0))
    half_hi = pl.BlockSpec((_BLK, _HC), lambda i: (i + _NB, 0))
    in_specs = [half, half_hi, half, half_hi,
                pl.BlockSpec((_BLK, 16), lambda i: (i, 0)),
                pl.BlockSpec((_BLK, 16), lambda i: (i, 0)),
                pl.BlockSpec((_C, _C), lambda i: (0, 0)),
                pl.BlockSpec((_C, _C), lambda i: (0, 0))]
    args = [aggf2, aggf2, aggb2, aggb2, degi, dego, w0, w1]
    if not first:
        in_specs += [pl.BlockSpec((_NB, 1, _C), lambda i: (0, 0, 0))] * 2
        args += [sums[0], sums[1]]
    if resid:
        in_specs.append(pl.BlockSpec((_BLK, _C), lambda i: (i, 0)))
        args.append(save)
    if last:
        out_specs = [half, half]
        out_shape = [jax.ShapeDtypeStruct((_NP, _HC), F32)] * 2
    else:
        out_specs = [half, half,
                     pl.BlockSpec((1, 1, _C), lambda i: (i, 0, 0)),
                     pl.BlockSpec((1, 1, _C), lambda i: (i, 0, 0))]
        out_shape = [jax.ShapeDtypeStruct((_NP, _HC), F32),
                     jax.ShapeDtypeStruct((_NP, _HC), F32),
                     jax.ShapeDtypeStruct((_NB, 1, _C), F32),
                     jax.ShapeDtypeStruct((_NB, 1, _C), F32)]
        if first or resid:
            out_specs.append(pl.BlockSpec((_BLK, _C), lambda i: (i, 0)))
            out_shape.append(jax.ShapeDtypeStruct((_NP, _C), F32))
    return pl.pallas_call(
        body, grid=(_NB,), in_specs=in_specs, out_specs=out_specs,
        out_shape=out_shape,
    )(*args)


def _tc_head_body(p_lo, p_hi, cnt, f1w, f1b, f2w, f2b, f3w, f3b, out):
    p = jnp.concatenate([p_lo[...], p_hi[...]], axis=1) / cnt[...]
    g = jnp.maximum(jnp.dot(p, f1w[...], precision=_PREC) + f1b[...], 0.0)
    g = jnp.maximum(jnp.dot(g, f2w[...], precision=_PREC) + f2b[...], 0.0)
    out[...] = jnp.dot(g, f3w[...], precision=_PREC) + f3b[...]


def _tc_head(p_lo, p_hi, cnt, f1w, f1b, f2w, f2b, f3w, f3b):
    return pl.pallas_call(
        _tc_head_body,
        out_shape=jax.ShapeDtypeStruct((_G, 1), F32),
    )(p_lo, p_hi, cnt, f1w, f1b, f2w, f2b, f3w, f3b)


def kernel(x, sources, targets, batch, counts, total, emb, conv_w, res_w,
           fc1_w, fc1_b, fc2_w, fc2_b, fc3_w, fc3_b):
    del total
    pad = _NP - _N
    xf = jnp.pad(x, (0, pad)).astype(F32)[:, None]
    sources = sources.astype(jnp.int32)
    targets = targets.astype(jnp.int32)
    srcf2 = jnp.concatenate([sources, sources + _NP])
    srcb2 = jnp.concatenate([targets, targets + _NP])
    eidx2 = jnp.concatenate([targets, sources])
    batchp = jnp.pad(batch.astype(jnp.int32), (0, pad))

    degflat = _sc_deg()(eidx2)
    degi, dego = degflat[:_NP], degflat[_NP:]

    h_lo, h_hi = _tc_embed(xf, emb)
    r2 = jnp.concatenate([h_lo, h_hi], axis=0)

    stage_w = [(conv_w[0], conv_w[1])]
    for i in range(4):
        for j in range(2):
            stage_w.append((res_w[i, j, 0], res_w[i, j, 1]))

    save = None
    sums = None
    out_final = None
    for k in range(9):
        aggf2, aggb2 = _sc_agg()(r2, srcf2, targets, srcb2, sources)
        first = k == 0
        resid = k in (2, 4, 6, 8)
        last = k == 8
        w0, w1 = stage_w[k]
        outs = _tc_stage(aggf2, aggb2, degi, dego, w0, w1, sums, save,
                         first, resid, last)
        if last:
            out_final = outs
        else:
            r_lo, r_hi, s_p, ss_p = outs[:4]
            if first or resid:
                save = outs[4]
            sums = (s_p, ss_p)
            r2 = jnp.concatenate([r_lo, r_hi], axis=0)

    h4 = jnp.concatenate(out_final, axis=0)
    pooledflat = _sc_pool()(h4, batchp)
    out2 = _tc_head(pooledflat[:_G], pooledflat[_G:],
                    counts[:, None], fc1_w, fc1_b[None, :],
                    fc2_w, fc2_b[None, :], fc3_w, fc3_b[None, :])
    return jnp.squeeze(out2, axis=-1)


# speedup vs baseline: 8.4693x; 1.7596x over previous
import functools

import jax
import jax.numpy as jnp
from jax import lax
from jax.experimental import pallas as pl
from jax.experimental.pallas import tpu as pltpu
from jax.experimental.pallas import tpu_sc as plsc

F32 = jnp.float32
_N = 50000
_E = 800000
_G = 128
_C = 64
_HC = 32
_NP = 50176
_BLK = 1024
_NB = _NP // _BLK
_NTILE = 16
_RPT = _NP // _NTILE
_EPT = _E // _NTILE
_EK = 2000
_ECH = _EPT // _EK
_EKA = 400
_ECHA = _EPT // _EKA
_KP = _RPT // 2
_EPS = 1e-5
_PREC = lax.Precision.HIGHEST


@functools.cache
def _sc_mesh():
    return plsc.VectorSubcoreMesh(core_axis_name="c", subcore_axis_name="s",
                                  num_cores=2, num_subcores=_NTILE)


def _sc_agg_body(r_lo, r_hi, src, tgt, af_lo, af_hi, ab_lo, ab_hi,
                 ig0, ig1, it0, it1, rows0, rows1, acc,
                 sig0, sig1, sit0, sit1, sg0, sg1):
    c = lax.axis_index("c")
    s = lax.axis_index("s")
    igs, its_, rws = (ig0, ig1), (it0, it1), (rows0, rows1)
    sigs, sits, sgs = (sig0, sig1), (sit0, sit1), (sg0, sg1)

    def direction(rh, garr, tarr, out):
        pltpu.sync_copy(rh.at[pl.ds(s * _RPT, _RPT)],
                        acc.at[pl.ds(s * _RPT, _RPT)])
        plsc.subcore_barrier()
        ebase = s * _EPT

        def issue_idx(k, p):
            off = ebase + k * _EKA
            pltpu.async_copy(garr.at[pl.ds(off, _EKA)], igs[p], sigs[p])
            pltpu.async_copy(tarr.at[pl.ds(off, _EKA)], its_[p], sits[p])

        def wait_idx(p):
            pltpu.make_async_copy(garr.at[pl.ds(ebase, _EKA)], igs[p],
                                  sigs[p]).wait()
            pltpu.make_async_copy(tarr.at[pl.ds(ebase, _EKA)], its_[p],
                                  sits[p]).wait()

        def issue_gather(p):
            pltpu.async_copy(rh.at[igs[p]], rws[p], sgs[p])

        def wait_gather(p):
            pltpu.make_async_copy(rh.at[igs[p]], rws[p], sgs[p]).wait()

        def process(p, k):
            q = 1 - p
            wait_gather(p)

            @pl.when(k + 1 < _ECHA)
            def _():
                wait_idx(q)
                issue_gather(q)

            pltpu.sync_copy(rws[p], acc.at[its_[p]], add=True)

            @pl.when(k + 2 < _ECHA)
            def _():
                issue_idx(k + 2, p)

        issue_idx(0, 0)
        issue_idx(1, 1)
        wait_idx(0)
        issue_gather(0)

        def body(k2, carry):
            k = 2 * k2
            process(0, k)
            process(1, k + 1)
            return carry

        lax.fori_loop(0, _ECHA // 2, body, 0)
        process(0, _ECHA - 1)
        plsc.subcore_barrier()
        pltpu.sync_copy(acc.at[pl.ds(s * _RPT, _RPT)],
                        out.at[pl.ds(s * _RPT, _RPT)])
        plsc.subcore_barrier()

    @pl.when(c == 0)
    def _():
        direction(r_lo, src, tgt, af_lo)
        direction(r_lo, tgt, src, ab_lo)

    @pl.when(c == 1)
    def _():
        direction(r_hi, src, tgt, af_hi)
        direction(r_hi, tgt, src, ab_hi)


@functools.cache
def _sc_agg():
    return pl.kernel(
        _sc_agg_body,
        out_type=[jax.ShapeDtypeStruct((_NP, _HC), F32)] * 4,
        mesh=_sc_mesh(),
        compiler_params=pltpu.CompilerParams(use_tc_tiling_on_sc=False),
        scratch_types=[
            pltpu.VMEM((_EKA,), jnp.int32),
            pltpu.VMEM((_EKA,), jnp.int32),
            pltpu.VMEM((_EKA,), jnp.int32),
            pltpu.VMEM((_EKA,), jnp.int32),
            pltpu.VMEM((_EKA, _HC), F32),
            pltpu.VMEM((_EKA, _HC), F32),
            pltpu.VMEM_SHARED((_NP, _HC), F32),
            pltpu.SemaphoreType.DMA,
            pltpu.SemaphoreType.DMA,
            pltpu.SemaphoreType.DMA,
            pltpu.SemaphoreType.DMA,
            pltpu.SemaphoreType.DMA,
            pltpu.SemaphoreType.DMA,
        ],
    )


def _sc_deg_body(tgt, src, deg_in, deg_out, idxbuf, ones, acc):
    c = lax.axis_index("c")
    s = lax.axis_index("s")

    def fill(i, carry):
        ones[i, :] = jnp.full((16,), 1.0, F32)
        return carry

    lax.fori_loop(0, _EK, fill, 0)
    pltpu.sync_copy(ones.at[pl.ds(0, _EK)], acc.at[pl.ds(s * _RPT, _EK)])
    pltpu.sync_copy(ones.at[pl.ds(0, _RPT - _EK)],
                    acc.at[pl.ds(s * _RPT + _EK, _RPT - _EK)])
    plsc.subcore_barrier()

    def run(earr, out):
        def chunk(k, carry):
            off = s * _EPT + k * _EK
            pltpu.sync_copy(earr.at[pl.ds(off, _EK)], idxbuf)
            pltpu.sync_copy(ones, acc.at[idxbuf], add=True)
            return carry

        lax.fori_loop(0, _ECH, chunk, 0)
        plsc.subcore_barrier()
        pltpu.sync_copy(acc.at[pl.ds(s * _RPT, _RPT)],
                        out.at[pl.ds(s * _RPT, _RPT)])

    @pl.when(c == 0)
    def _():
        run(tgt, deg_in)

    @pl.when(c == 1)
    def _():
        run(src, deg_out)


@functools.cache
def _sc_deg():
    return pl.kernel(
        _sc_deg_body,
        out_type=[jax.ShapeDtypeStruct((_NP, 16), F32)] * 2,
        mesh=_sc_mesh(),
        compiler_params=pltpu.CompilerParams(use_tc_tiling_on_sc=False),
        scratch_types=[
            pltpu.VMEM((_EK,), jnp.int32),
            pltpu.VMEM((_EK, 16), F32),
            pltpu.VMEM_SHARED((_NP, 16), F32),
        ],
    )


def _sc_pool_body(h_lo, h_hi, batchp, p_lo, p_hi, idxbuf, rows, acc):
    c = lax.axis_index("c")
    s = lax.axis_index("s")

    @pl.when(s == 0)
    def _init():
        def fill(i, carry):
            rows[i, 0:16] = jnp.zeros((16,), F32)
            rows[i, 16:32] = jnp.zeros((16,), F32)
            return carry

        lax.fori_loop(0, _G, fill, 0)
        pltpu.sync_copy(rows.at[pl.ds(0, _G)], acc)

    plsc.subcore_barrier()

    def run(harr, out):
        def chunk(k, carry):
            roff = s * _RPT + k * _KP
            pltpu.sync_copy(harr.at[pl.ds(roff, _KP)], rows)
            pltpu.sync_copy(batchp.at[pl.ds(roff, _KP)], idxbuf)
            pltpu.sync_copy(rows, acc.at[idxbuf], add=True)
            return carry

        lax.fori_loop(0, 2, chunk, 0)
        plsc.subcore_barrier()

        @pl.when(s == 0)
        def _out():
            pltpu.sync_copy(acc, out)

    @pl.when(c == 0)
    def _():
        run(h_lo, p_lo)

    @pl.when(c == 1)
    def _():
        run(h_hi, p_hi)


@functools.cache
def _sc_pool():
    return pl.kernel(
        _sc_pool_body,
        out_type=[jax.ShapeDtypeStruct((_G, _HC), F32)] * 2,
        mesh=_sc_mesh(),
        compiler_params=pltpu.CompilerParams(use_tc_tiling_on_sc=False),
        scratch_types=[
            pltpu.VMEM((_KP,), jnp.int32),
            pltpu.VMEM((_KP, _HC), F32),
            pltpu.VMEM_SHARED((_G, _HC), F32),
        ],
    )


def _tc_embed_body(xf, emb, h_lo, h_hi):
    i = pl.program_id(0)
    xv = xf[...]
    e = emb[...]
    h = jnp.zeros((_BLK, _C), F32)
    for k in range(7):
        h = h + jnp.where(xv == float(k), 1.0, 0.0) * e[k][None, :]
    rowid = i * _BLK + lax.broadcasted_iota(jnp.int32, (_BLK, 1), 0)
    h = jnp.where(rowid < _N, h, 0.0)
    h_lo[...] = h[:, :_HC]
    h_hi[...] = h[:, _HC:]


def _tc_embed(xf, emb):
    return pl.pallas_call(
        _tc_embed_body,
        grid=(_NB,),
        in_specs=[
            pl.BlockSpec((_BLK, 1), lambda i: (i, 0)),
            pl.BlockSpec((7, _C), lambda i: (0, 0)),
        ],
        out_specs=[
            pl.BlockSpec((_BLK, _HC), lambda i: (i, 0)),
            pl.BlockSpec((_BLK, _HC), lambda i: (i, 0)),
        ],
        out_shape=[jax.ShapeDtypeStruct((_NP, _HC), F32),
                   jax.ShapeDtypeStruct((_NP, _HC), F32)],
    )(xf, emb)


def _tc_stage(aggs, degi, dego, w0, w1, sums, save, first, resid, last):
    def body(*refs):
        it = iter(refs)
        aggf_lo, aggf_hi, aggb_lo, aggb_hi, degi_r, dego_r, w0_r, w1_r = (
            next(it) for _ in range(8))
        sum_r = sumsq_r = save_r = None
        if not first:
            sum_r, sumsq_r = next(it), next(it)
        if resid:
            save_r = next(it)
        if last:
            h_lo, h_hi = next(it), next(it)
        else:
            r_lo, r_hi, sum_o, sumsq_o = (next(it) for _ in range(4))
            save_o = next(it) if (first or resid) else None

        i = pl.program_id(0)
        aggf = jnp.concatenate([aggf_lo[...], aggf_hi[...]], axis=1)
        aggb = jnp.concatenate([aggb_lo[...], aggb_hi[...]], axis=1)
        norm = 1.0 / degi_r[:, 0:1]
        normt = 1.0 / dego_r[:, 0:1]
        w0v, w1v = w0_r[...], w1_r[...]
        if not first:
            m = jnp.sum(sum_r[...], axis=(0, 1)) * (1.0 / _N)
            var = jnp.sum(sumsq_r[...], axis=(0, 1)) * (1.0 / _N) - m * m
            inv = lax.rsqrt(var + _EPS)
            aggf = aggf * inv[None, :]
            aggb = aggb * inv[None, :]
        h = (jnp.dot(norm * aggf, w0v, precision=_PREC) +
             jnp.dot(normt * aggb, w1v, precision=_PREC))
        if not first:
            bias = jnp.dot((m * inv)[None, :], w0v + w1v, precision=_PREC)
            h = h - bias
        if resid:
            h = h + save_r[...]
        rowid = i * _BLK + lax.broadcasted_iota(jnp.int32, (_BLK, 1), 0)
        h = jnp.where(rowid < _N, h, 0.0)
        if last:
            h_lo[...] = h[:, :_HC]
            h_hi[...] = h[:, _HC:]
        else:
            if save_o is not None:
                save_o[...] = h
            r = jnp.maximum(h, 0.0)
            r_lo[...] = r[:, :_HC]
            r_hi[...] = r[:, _HC:]
            sum_o[...] = jnp.sum(r, axis=0, keepdims=True)[None]
            sumsq_o[...] = jnp.sum(r * r, axis=0, keepdims=True)[None]

    half = pl.BlockSpec((_BLK, _HC), lambda i: (i, 0))
    in_specs = [half, half, half, half,
                pl.BlockSpec((_BLK, 16), lambda i: (i, 0)),
                pl.BlockSpec((_BLK, 16), lambda i: (i, 0)),
                pl.BlockSpec((_C, _C), lambda i: (0, 0)),
                pl.BlockSpec((_C, _C), lambda i: (0, 0))]
    args = list(aggs) + [degi, dego, w0, w1]
    if not first:
        in_specs += [pl.BlockSpec((_NB, 1, _C), lambda i: (0, 0, 0))] * 2
        args += [sums[0], sums[1]]
    if resid:
        in_specs.append(pl.BlockSpec((_BLK, _C), lambda i: (i, 0)))
        args.append(save)
    if last:
        out_specs = [half, half]
        out_shape = [jax.ShapeDtypeStruct((_NP, _HC), F32)] * 2
    else:
        out_specs = [half, half,
                     pl.BlockSpec((1, 1, _C), lambda i: (i, 0, 0)),
                     pl.BlockSpec((1, 1, _C), lambda i: (i, 0, 0))]
        out_shape = [jax.ShapeDtypeStruct((_NP, _HC), F32),
                     jax.ShapeDtypeStruct((_NP, _HC), F32),
                     jax.ShapeDtypeStruct((_NB, 1, _C), F32),
                     jax.ShapeDtypeStruct((_NB, 1, _C), F32)]
        if first or resid:
            out_specs.append(pl.BlockSpec((_BLK, _C), lambda i: (i, 0)))
            out_shape.append(jax.ShapeDtypeStruct((_NP, _C), F32))
    return pl.pallas_call(
        body, grid=(_NB,), in_specs=in_specs, out_specs=out_specs,
        out_shape=out_shape,
    )(*args)


def _tc_head_body(p_lo, p_hi, cnt, f1w, f1b, f2w, f2b, f3w, f3b, out):
    p = jnp.concatenate([p_lo[...], p_hi[...]], axis=1) / cnt[...]
    g = jnp.maximum(jnp.dot(p, f1w[...], precision=_PREC) + f1b[...], 0.0)
    g = jnp.maximum(jnp.dot(g, f2w[...], precision=_PREC) + f2b[...], 0.0)
    out[...] = jnp.dot(g, f3w[...], precision=_PREC) + f3b[...]


def _tc_head(p_lo, p_hi, cnt, f1w, f1b, f2w, f2b, f3w, f3b):
    return pl.pallas_call(
        _tc_head_body,
        out_shape=jax.ShapeDtypeStruct((_G, 1), F32),
    )(p_lo, p_hi, cnt, f1w, f1b, f2w, f2b, f3w, f3b)


def kernel(x, sources, targets, batch, counts, total, emb, conv_w, res_w,
           fc1_w, fc1_b, fc2_w, fc2_b, fc3_w, fc3_b):
    del total
    pad = _NP - _N
    xf = jnp.pad(x, (0, pad)).astype(F32)[:, None]
    sources = sources.astype(jnp.int32)
    targets = targets.astype(jnp.int32)
    batchp = jnp.pad(batch.astype(jnp.int32), (0, pad))

    degi, dego = _sc_deg()(targets, sources)
    r_lo, r_hi = _tc_embed(xf, emb)

    stage_w = [(conv_w[0], conv_w[1])]
    for i in range(4):
        for j in range(2):
            stage_w.append((res_w[i, j, 0], res_w[i, j, 1]))

    save = None
    sums = None
    out_final = None
    for k in range(9):
        aggs = _sc_agg()(r_lo, r_hi, sources, targets)
        first = k == 0
        resid = k in (2, 4, 6, 8)
        last = k == 8
        w0, w1 = stage_w[k]
        outs = _tc_stage(aggs, degi, dego, w0, w1, sums, save,
                         first, resid, last)
        if last:
            out_final = outs
        else:
            r_lo, r_hi, s_p, ss_p = outs[:4]
            if first or resid:
                save = outs[4]
            sums = (s_p, ss_p)

    p_lo, p_hi = _sc_pool()(out_final[0], out_final[1], batchp)
    out2 = _tc_head(p_lo, p_hi, counts[:, None], fc1_w, fc1_b[None, :],
                    fc2_w, fc2_b[None, :], fc3_w, fc3_b[None, :])
    return jnp.squeeze(out2, axis=-1)


# speedup vs baseline: 8.8021x; 1.0393x over previous
import functools

import jax
import jax.numpy as jnp
from jax import lax
from jax.experimental import pallas as pl
from jax.experimental.pallas import tpu as pltpu
from jax.experimental.pallas import tpu_sc as plsc

F32 = jnp.float32
_N = 50000
_E = 800000
_G = 128
_C = 64
_HC = 32
_NP = 50176
_BLK = 3584
_NB = _NP // _BLK
_NTILE = 16
_RPT = _NP // _NTILE
_EPT = _E // _NTILE
_EK = 2000
_ECH = _EPT // _EK
_EKA = 400
_ECHA = _EPT // _EKA
_KP = _RPT // 2
_EPS = 1e-5
_PREC = lax.Precision.HIGHEST


@functools.cache
def _sc_mesh():
    return plsc.VectorSubcoreMesh(core_axis_name="c", subcore_axis_name="s",
                                  num_cores=2, num_subcores=_NTILE)


def _sc_agg_body(r_lo, r_hi, src, tgt, af_lo, af_hi, ab_lo, ab_hi,
                 ig0, ig1, it0, it1, rows0, rows1, acc,
                 sig0, sig1, sit0, sit1, sg0, sg1):
    c = lax.axis_index("c")
    s = lax.axis_index("s")
    igs, its_, rws = (ig0, ig1), (it0, it1), (rows0, rows1)
    sigs, sits, sgs = (sig0, sig1), (sit0, sit1), (sg0, sg1)

    def direction(rh, garr, tarr, out):
        pltpu.sync_copy(rh.at[pl.ds(s * _RPT, _RPT)],
                        acc.at[pl.ds(s * _RPT, _RPT)])
        plsc.subcore_barrier()
        ebase = s * _EPT

        def issue_idx(k, p):
            off = ebase + k * _EKA
            pltpu.async_copy(garr.at[pl.ds(off, _EKA)], igs[p], sigs[p])
            pltpu.async_copy(tarr.at[pl.ds(off, _EKA)], its_[p], sits[p])

        def wait_idx(p):
            pltpu.make_async_copy(garr.at[pl.ds(ebase, _EKA)], igs[p],
                                  sigs[p]).wait()
            pltpu.make_async_copy(tarr.at[pl.ds(ebase, _EKA)], its_[p],
                                  sits[p]).wait()

        def issue_gather(p):
            pltpu.async_copy(rh.at[igs[p]], rws[p], sgs[p])

        def wait_gather(p):
            pltpu.make_async_copy(rh.at[igs[p]], rws[p], sgs[p]).wait()

        def process(p, k):
            q = 1 - p
            wait_gather(p)

            @pl.when(k + 1 < _ECHA)
            def _():
                wait_idx(q)
                issue_gather(q)

            pltpu.sync_copy(rws[p], acc.at[its_[p]], add=True)

            @pl.when(k + 2 < _ECHA)
            def _():
                issue_idx(k + 2, p)

        issue_idx(0, 0)
        issue_idx(1, 1)
        wait_idx(0)
        issue_gather(0)

        def body(k2, carry):
            k = 2 * k2
            process(0, k)
            process(1, k + 1)
            return carry

        lax.fori_loop(0, _ECHA // 2, body, 0)
        process(0, _ECHA - 1)
        plsc.subcore_barrier()
        pltpu.sync_copy(acc.at[pl.ds(s * _RPT, _RPT)],
                        out.at[pl.ds(s * _RPT, _RPT)])
        plsc.subcore_barrier()

    @pl.when(c == 0)
    def _():
        direction(r_lo, src, tgt, af_lo)
        direction(r_lo, tgt, src, ab_lo)

    @pl.when(c == 1)
    def _():
        direction(r_hi, src, tgt, af_hi)
        direction(r_hi, tgt, src, ab_hi)


@functools.cache
def _sc_agg():
    return pl.kernel(
        _sc_agg_body,
        out_type=[jax.ShapeDtypeStruct((_NP, _HC), F32)] * 4,
        mesh=_sc_mesh(),
        compiler_params=pltpu.CompilerParams(use_tc_tiling_on_sc=False),
        scratch_types=[
            pltpu.VMEM((_EKA,), jnp.int32),
            pltpu.VMEM((_EKA,), jnp.int32),
            pltpu.VMEM((_EKA,), jnp.int32),
            pltpu.VMEM((_EKA,), jnp.int32),
            pltpu.VMEM((_EKA, _HC), F32),
            pltpu.VMEM((_EKA, _HC), F32),
            pltpu.VMEM_SHARED((_NP, _HC), F32),
            pltpu.SemaphoreType.DMA,
            pltpu.SemaphoreType.DMA,
            pltpu.SemaphoreType.DMA,
            pltpu.SemaphoreType.DMA,
            pltpu.SemaphoreType.DMA,
            pltpu.SemaphoreType.DMA,
        ],
    )


def _sc_deg_body(tgt, src, deg_in, deg_out, idxbuf, ones, acc):
    c = lax.axis_index("c")
    s = lax.axis_index("s")

    def fill(i, carry):
        ones[i, :] = jnp.full((16,), 1.0, F32)
        return carry

    lax.fori_loop(0, _EK, fill, 0)
    pltpu.sync_copy(ones.at[pl.ds(0, _EK)], acc.at[pl.ds(s * _RPT, _EK)])
    pltpu.sync_copy(ones.at[pl.ds(0, _RPT - _EK)],
                    acc.at[pl.ds(s * _RPT + _EK, _RPT - _EK)])
    plsc.subcore_barrier()

    def run(earr, out):
        def chunk(k, carry):
            off = s * _EPT + k * _EK
            pltpu.sync_copy(earr.at[pl.ds(off, _EK)], idxbuf)
            pltpu.sync_copy(ones, acc.at[idxbuf], add=True)
            return carry

        lax.fori_loop(0, _ECH, chunk, 0)
        plsc.subcore_barrier()
        pltpu.sync_copy(acc.at[pl.ds(s * _RPT, _RPT)],
                        out.at[pl.ds(s * _RPT, _RPT)])

    @pl.when(c == 0)
    def _():
        run(tgt, deg_in)

    @pl.when(c == 1)
    def _():
        run(src, deg_out)


@functools.cache
def _sc_deg():
    return pl.kernel(
        _sc_deg_body,
        out_type=[jax.ShapeDtypeStruct((_NP, 16), F32)] * 2,
        mesh=_sc_mesh(),
        compiler_params=pltpu.CompilerParams(use_tc_tiling_on_sc=False),
        scratch_types=[
            pltpu.VMEM((_EK,), jnp.int32),
            pltpu.VMEM((_EK, 16), F32),
            pltpu.VMEM_SHARED((_NP, 16), F32),
        ],
    )


def _sc_pool_body(h_lo, h_hi, batchp, p_lo, p_hi, idxbuf, rows, acc):
    c = lax.axis_index("c")
    s = lax.axis_index("s")

    @pl.when(s == 0)
    def _init():
        def fill(i, carry):
            rows[i, 0:16] = jnp.zeros((16,), F32)
            rows[i, 16:32] = jnp.zeros((16,), F32)
            return carry

        lax.fori_loop(0, _G, fill, 0)
        pltpu.sync_copy(rows.at[pl.ds(0, _G)], acc)

    plsc.subcore_barrier()

    def run(harr, out):
        def chunk(k, carry):
            roff = s * _RPT + k * _KP
            pltpu.sync_copy(harr.at[pl.ds(roff, _KP)], rows)
            pltpu.sync_copy(batchp.at[pl.ds(roff, _KP)], idxbuf)
            pltpu.sync_copy(rows, acc.at[idxbuf], add=True)
            return carry

        lax.fori_loop(0, 2, chunk, 0)
        plsc.subcore_barrier()

        @pl.when(s == 0)
        def _out():
            pltpu.sync_copy(acc, out)

    @pl.when(c == 0)
    def _():
        run(h_lo, p_lo)

    @pl.when(c == 1)
    def _():
        run(h_hi, p_hi)


@functools.cache
def _sc_pool():
    return pl.kernel(
        _sc_pool_body,
        out_type=[jax.ShapeDtypeStruct((_G, _HC), F32)] * 2,
        mesh=_sc_mesh(),
        compiler_params=pltpu.CompilerParams(use_tc_tiling_on_sc=False),
        scratch_types=[
            pltpu.VMEM((_KP,), jnp.int32),
            pltpu.VMEM((_KP, _HC), F32),
            pltpu.VMEM_SHARED((_G, _HC), F32),
        ],
    )


def _tc_embed_body(xf, emb, h_lo, h_hi):
    i = pl.program_id(0)
    xv = xf[...]
    e = emb[...]
    h = jnp.zeros((_BLK, _C), F32)
    for k in range(7):
        h = h + jnp.where(xv == float(k), 1.0, 0.0) * e[k][None, :]
    rowid = i * _BLK + lax.broadcasted_iota(jnp.int32, (_BLK, 1), 0)
    h = jnp.where(rowid < _N, h, 0.0)
    h_lo[...] = h[:, :_HC]
    h_hi[...] = h[:, _HC:]


def _tc_embed(xf, emb):
    return pl.pallas_call(
        _tc_embed_body,
        grid=(_NB,),
        in_specs=[
            pl.BlockSpec((_BLK, 1), lambda i: (i, 0)),
            pl.BlockSpec((7, _C), lambda i: (0, 0)),
        ],
        out_specs=[
            pl.BlockSpec((_BLK, _HC), lambda i: (i, 0)),
            pl.BlockSpec((_BLK, _HC), lambda i: (i, 0)),
        ],
        out_shape=[jax.ShapeDtypeStruct((_NP, _HC), F32),
                   jax.ShapeDtypeStruct((_NP, _HC), F32)],
    )(xf, emb)


def _tc_stage(aggs, degi, dego, w0, w1, sums, save, first, resid, last):
    def body(*refs):
        it = iter(refs)
        aggf_lo, aggf_hi, aggb_lo, aggb_hi, degi_r, dego_r, w0_r, w1_r = (
            next(it) for _ in range(8))
        sum_r = sumsq_r = save_r = None
        if not first:
            sum_r, sumsq_r = next(it), next(it)
        if resid:
            save_r = next(it)
        if last:
            h_lo, h_hi = next(it), next(it)
        else:
            r_lo, r_hi, sum_o, sumsq_o = (next(it) for _ in range(4))
            save_o = next(it) if (first or resid) else None

        i = pl.program_id(0)
        aggf = jnp.concatenate([aggf_lo[...], aggf_hi[...]], axis=1)
        aggb = jnp.concatenate([aggb_lo[...], aggb_hi[...]], axis=1)
        norm = 1.0 / degi_r[:, 0:1]
        normt = 1.0 / dego_r[:, 0:1]
        w0v, w1v = w0_r[...], w1_r[...]
        if not first:
            m = jnp.sum(sum_r[...], axis=(0, 1)) * (1.0 / _N)
            var = jnp.sum(sumsq_r[...], axis=(0, 1)) * (1.0 / _N) - m * m
            inv = lax.rsqrt(var + _EPS)
            aggf = aggf * inv[None, :]
            aggb = aggb * inv[None, :]
        h = (jnp.dot(norm * aggf, w0v, precision=_PREC) +
             jnp.dot(normt * aggb, w1v, precision=_PREC))
        if not first:
            bias = jnp.dot((m * inv)[None, :], w0v + w1v, precision=_PREC)
            h = h - bias
        if resid:
            h = h + save_r[...]
        rowid = i * _BLK + lax.broadcasted_iota(jnp.int32, (_BLK, 1), 0)
        h = jnp.where(rowid < _N, h, 0.0)
        if last:
            h_lo[...] = h[:, :_HC]
            h_hi[...] = h[:, _HC:]
        else:
            if save_o is not None:
                save_o[...] = h
            r = jnp.maximum(h, 0.0)
            r_lo[...] = r[:, :_HC]
            r_hi[...] = r[:, _HC:]
            sum_o[...] = jnp.sum(r, axis=0, keepdims=True)[None]
            sumsq_o[...] = jnp.sum(r * r, axis=0, keepdims=True)[None]

    half = pl.BlockSpec((_BLK, _HC), lambda i: (i, 0))
    in_specs = [half, half, half, half,
                pl.BlockSpec((_BLK, 16), lambda i: (i, 0)),
                pl.BlockSpec((_BLK, 16), lambda i: (i, 0)),
                pl.BlockSpec((_C, _C), lambda i: (0, 0)),
                pl.BlockSpec((_C, _C), lambda i: (0, 0))]
    args = list(aggs) + [degi, dego, w0, w1]
    if not first:
        in_specs += [pl.BlockSpec((_NB, 1, _C), lambda i: (0, 0, 0))] * 2
        args += [sums[0], sums[1]]
    if resid:
        in_specs.append(pl.BlockSpec((_BLK, _C), lambda i: (i, 0)))
        args.append(save)
    if last:
        out_specs = [half, half]
        out_shape = [jax.ShapeDtypeStruct((_NP, _HC), F32)] * 2
    else:
        out_specs = [half, half,
                     pl.BlockSpec((1, 1, _C), lambda i: (i, 0, 0)),
                     pl.BlockSpec((1, 1, _C), lambda i: (i, 0, 0))]
        out_shape = [jax.ShapeDtypeStruct((_NP, _HC), F32),
                     jax.ShapeDtypeStruct((_NP, _HC), F32),
                     jax.ShapeDtypeStruct((_NB, 1, _C), F32),
                     jax.ShapeDtypeStruct((_NB, 1, _C), F32)]
        if first or resid:
            out_specs.append(pl.BlockSpec((_BLK, _C), lambda i: (i, 0)))
            out_shape.append(jax.ShapeDtypeStruct((_NP, _C), F32))
    return pl.pallas_call(
        body, grid=(_NB,), in_specs=in_specs, out_specs=out_specs,
        out_shape=out_shape,
    )(*args)


def _tc_head_body(p_lo, p_hi, cnt, f1w, f1b, f2w, f2b, f3w, f3b, out):
    p = jnp.concatenate([p_lo[...], p_hi[...]], axis=1) / cnt[...]
    g = jnp.maximum(jnp.dot(p, f1w[...], precision=_PREC) + f1b[...], 0.0)
    g = jnp.maximum(jnp.dot(g, f2w[...], precision=_PREC) + f2b[...], 0.0)
    out[...] = jnp.dot(g, f3w[...], precision=_PREC) + f3b[...]


def _tc_head(p_lo, p_hi, cnt, f1w, f1b, f2w, f2b, f3w, f3b):
    return pl.pallas_call(
        _tc_head_body,
        out_shape=jax.ShapeDtypeStruct((_G, 1), F32),
    )(p_lo, p_hi, cnt, f1w, f1b, f2w, f2b, f3w, f3b)


def kernel(x, sources, targets, batch, counts, total, emb, conv_w, res_w,
           fc1_w, fc1_b, fc2_w, fc2_b, fc3_w, fc3_b):
    del total
    pad = _NP - _N
    xf = jnp.pad(x, (0, pad)).astype(F32)[:, None]
    sources = sources.astype(jnp.int32)
    targets = targets.astype(jnp.int32)
    batchp = jnp.pad(batch.astype(jnp.int32), (0, pad))

    degi, dego = _sc_deg()(targets, sources)
    r_lo, r_hi = _tc_embed(xf, emb)

    stage_w = [(conv_w[0], conv_w[1])]
    for i in range(4):
        for j in range(2):
            stage_w.append((res_w[i, j, 0], res_w[i, j, 1]))

    save = None
    sums = None
    out_final = None
    for k in range(9):
        aggs = _sc_agg()(r_lo, r_hi, sources, targets)
        first = k == 0
        resid = k in (2, 4, 6, 8)
        last = k == 8
        w0, w1 = stage_w[k]
        outs = _tc_stage(aggs, degi, dego, w0, w1, sums, save,
                         first, resid, last)
        if last:
            out_final = outs
        else:
            r_lo, r_hi, s_p, ss_p = outs[:4]
            if first or resid:
                save = outs[4]
            sums = (s_p, ss_p)

    p_lo, p_hi = _sc_pool()(out_final[0], out_final[1], batchp)
    out2 = _tc_head(p_lo, p_hi, counts[:, None], fc1_w, fc1_b[None, :],
                    fc2_w, fc2_b[None, :], fc3_w, fc3_b[None, :])
    return jnp.squeeze(out2, axis=-1)


# speedup vs baseline: 8.8256x; 1.0027x over previous
import functools

import jax
import jax.numpy as jnp
from jax import lax
from jax.experimental import pallas as pl
from jax.experimental.pallas import tpu as pltpu
from jax.experimental.pallas import tpu_sc as plsc

F32 = jnp.float32
_N = 50000
_E = 800000
_G = 128
_C = 64
_HC = 32
_NP = 50176
_BLK = 3584
_NB = _NP // _BLK
_NTILE = 16
_RPT = _NP // _NTILE
_EPT = _E // _NTILE
_EK = 2000
_ECH = _EPT // _EK
_EKA = 400
_ECHA = _EPT // _EKA
_KP = _RPT // 2
_EPS = 1e-5
_PREC = lax.Precision.HIGHEST


@functools.cache
def _sc_mesh():
    return plsc.VectorSubcoreMesh(core_axis_name="c", subcore_axis_name="s",
                                  num_cores=2, num_subcores=_NTILE)


def _sc_agg_body(r_lo, r_hi, src, tgt, af_lo, af_hi, ab_lo, ab_hi,
                 ig0, ig1, it00, it01, it10, it11, rows0, rows1, acc,
                 zidx, zrows,
                 sig0, sig1, sit0, sit1, sg0, sg1, ss0, ss1):
    c = lax.axis_index("c")
    s = lax.axis_index("s")
    zidx[...] = lax.iota(jnp.int32, 16)
    for _zi in range(16):
        zrows[_zi, 0:16] = jnp.zeros((16,), F32)
        zrows[_zi, 16:32] = jnp.zeros((16,), F32)
    igs, rws = (ig0, ig1), (rows0, rows1)
    its_ = ((it00, it01), (it10, it11))
    sigs, sits, sgs, sss = (sig0, sig1), (sit0, sit1), (sg0, sg1), (ss0, ss1)

    def direction(rh, garr, tarr, out):
        pltpu.sync_copy(rh.at[pl.ds(s * _RPT, _RPT)],
                        acc.at[pl.ds(s * _RPT, _RPT)])
        plsc.subcore_barrier()
        ebase = s * _EPT

        def issue_idx(k, p, g):
            off = ebase + k * _EKA
            pltpu.async_copy(garr.at[pl.ds(off, _EKA)], igs[p], sigs[p])
            pltpu.async_copy(tarr.at[pl.ds(off, _EKA)], its_[p][g], sits[p])

        def wait_idx(p, g):
            pltpu.make_async_copy(garr.at[pl.ds(ebase, _EKA)], igs[p],
                                  sigs[p]).wait()
            pltpu.make_async_copy(tarr.at[pl.ds(ebase, _EKA)], its_[p][g],
                                  sits[p]).wait()

        def issue_gather(p):
            pltpu.async_copy(rh.at[igs[p]], rws[p], sgs[p])

        def wait_gather(p):
            pltpu.make_async_copy(rh.at[igs[p]], rws[p], sgs[p]).wait()

        def issue_scatter(p, g):
            pltpu.async_copy(rws[p], acc.at[its_[p][g]], sss[p], add=True)

        def wait_scatter(p, g):
            pltpu.make_async_copy(rws[p], acc.at[its_[p][g]], sss[p]).wait()

        def process(p, g, gn, gp, k):
            q = 1 - p
            wait_gather(p)

            @pl.when(k + 1 < _ECHA)
            def _():
                wait_idx(q, gn)

                @pl.when(k >= 1)
                def _():
                    wait_scatter(q, gp)

                issue_gather(q)

            issue_scatter(p, g)

            @pl.when(k + 2 < _ECHA)
            def _():
                issue_idx(k + 2, p, 1 - g)

        issue_idx(0, 0, 0)
        issue_idx(1, 1, 0)
        wait_idx(0, 0)
        issue_gather(0)

        def body(k2, carry):
            k = 4 * k2
            process(0, 0, 0, 1, k)
            process(1, 0, 1, 0, k + 1)
            process(0, 1, 1, 0, k + 2)
            process(1, 1, 0, 1, k + 3)
            return carry

        lax.fori_loop(0, _ECHA // 4, body, 0)
        process(0, 0, 0, 1, _ECHA - 1)
        wait_scatter(1, 1)
        wait_scatter(0, 0)
        pltpu.sync_copy(zrows, acc.at[zidx], add=True)
        plsc.subcore_barrier()
        pltpu.sync_copy(acc.at[pl.ds(s * _RPT, _RPT)],
                        out.at[pl.ds(s * _RPT, _RPT)])
        plsc.subcore_barrier()

    @pl.when(c == 0)
    def _():
        direction(r_lo, src, tgt, af_lo)
        direction(r_lo, tgt, src, ab_lo)

    @pl.when(c == 1)
    def _():
        direction(r_hi, src, tgt, af_hi)
        direction(r_hi, tgt, src, ab_hi)


@functools.cache
def _sc_agg():
    return pl.kernel(
        _sc_agg_body,
        out_type=[jax.ShapeDtypeStruct((_NP, _HC), F32)] * 4,
        mesh=_sc_mesh(),
        compiler_params=pltpu.CompilerParams(use_tc_tiling_on_sc=False),
        scratch_types=[
            pltpu.VMEM((_EKA,), jnp.int32),
            pltpu.VMEM((_EKA,), jnp.int32),
            pltpu.VMEM((_EKA,), jnp.int32),
            pltpu.VMEM((_EKA,), jnp.int32),
            pltpu.VMEM((_EKA,), jnp.int32),
            pltpu.VMEM((_EKA,), jnp.int32),
            pltpu.VMEM((_EKA, _HC), F32),
            pltpu.VMEM((_EKA, _HC), F32),
            pltpu.VMEM_SHARED((_NP, _HC), F32),
            pltpu.VMEM((16,), jnp.int32),
            pltpu.VMEM((16, _HC), F32),
            pltpu.SemaphoreType.DMA,
            pltpu.SemaphoreType.DMA,
            pltpu.SemaphoreType.DMA,
            pltpu.SemaphoreType.DMA,
            pltpu.SemaphoreType.DMA,
            pltpu.SemaphoreType.DMA,
            pltpu.SemaphoreType.DMA,
            pltpu.SemaphoreType.DMA,
        ],
    )


def _sc_deg_body(tgt, src, deg_in, deg_out, idxbuf, ones, acc):
    c = lax.axis_index("c")
    s = lax.axis_index("s")

    def fill(i, carry):
        ones[i, :] = jnp.full((16,), 1.0, F32)
        return carry

    lax.fori_loop(0, _EK, fill, 0)
    pltpu.sync_copy(ones.at[pl.ds(0, _EK)], acc.at[pl.ds(s * _RPT, _EK)])
    pltpu.sync_copy(ones.at[pl.ds(0, _RPT - _EK)],
                    acc.at[pl.ds(s * _RPT + _EK, _RPT - _EK)])
    plsc.subcore_barrier()

    def run(earr, out):
        def chunk(k, carry):
            off = s * _EPT + k * _EK
            pltpu.sync_copy(earr.at[pl.ds(off, _EK)], idxbuf)
            pltpu.sync_copy(ones, acc.at[idxbuf], add=True)
            return carry

        lax.fori_loop(0, _ECH, chunk, 0)
        plsc.subcore_barrier()
        pltpu.sync_copy(acc.at[pl.ds(s * _RPT, _RPT)],
                        out.at[pl.ds(s * _RPT, _RPT)])

    @pl.when(c == 0)
    def _():
        run(tgt, deg_in)

    @pl.when(c == 1)
    def _():
        run(src, deg_out)


@functools.cache
def _sc_deg():
    return pl.kernel(
        _sc_deg_body,
        out_type=[jax.ShapeDtypeStruct((_NP, 16), F32)] * 2,
        mesh=_sc_mesh(),
        compiler_params=pltpu.CompilerParams(use_tc_tiling_on_sc=False),
        scratch_types=[
            pltpu.VMEM((_EK,), jnp.int32),
            pltpu.VMEM((_EK, 16), F32),
            pltpu.VMEM_SHARED((_NP, 16), F32),
        ],
    )


def _sc_pool_body(h_lo, h_hi, batchp, p_lo, p_hi, idxbuf, rows, acc):
    c = lax.axis_index("c")
    s = lax.axis_index("s")

    @pl.when(s == 0)
    def _init():
        def fill(i, carry):
            rows[i, 0:16] = jnp.zeros((16,), F32)
            rows[i, 16:32] = jnp.zeros((16,), F32)
            return carry

        lax.fori_loop(0, _G, fill, 0)
        pltpu.sync_copy(rows.at[pl.ds(0, _G)], acc)

    plsc.subcore_barrier()

    def run(harr, out):
        def chunk(k, carry):
            roff = s * _RPT + k * _KP
            pltpu.sync_copy(harr.at[pl.ds(roff, _KP)], rows)
            pltpu.sync_copy(batchp.at[pl.ds(roff, _KP)], idxbuf)
            pltpu.sync_copy(rows, acc.at[idxbuf], add=True)
            return carry

        lax.fori_loop(0, 2, chunk, 0)
        plsc.subcore_barrier()

        @pl.when(s == 0)
        def _out():
            pltpu.sync_copy(acc, out)

    @pl.when(c == 0)
    def _():
        run(h_lo, p_lo)

    @pl.when(c == 1)
    def _():
        run(h_hi, p_hi)


@functools.cache
def _sc_pool():
    return pl.kernel(
        _sc_pool_body,
        out_type=[jax.ShapeDtypeStruct((_G, _HC), F32)] * 2,
        mesh=_sc_mesh(),
        compiler_params=pltpu.CompilerParams(use_tc_tiling_on_sc=False),
        scratch_types=[
            pltpu.VMEM((_KP,), jnp.int32),
            pltpu.VMEM((_KP, _HC), F32),
            pltpu.VMEM_SHARED((_G, _HC), F32),
        ],
    )


def _tc_embed_body(xf, emb, h_lo, h_hi):
    i = pl.program_id(0)
    xv = xf[...]
    e = emb[...]
    h = jnp.zeros((_BLK, _C), F32)
    for k in range(7):
        h = h + jnp.where(xv == float(k), 1.0, 0.0) * e[k][None, :]
    rowid = i * _BLK + lax.broadcasted_iota(jnp.int32, (_BLK, 1), 0)
    h = jnp.where(rowid < _N, h, 0.0)
    h_lo[...] = h[:, :_HC]
    h_hi[...] = h[:, _HC:]


def _tc_embed(xf, emb):
    return pl.pallas_call(
        _tc_embed_body,
        grid=(_NB,),
        in_specs=[
            pl.BlockSpec((_BLK, 1), lambda i: (i, 0)),
            pl.BlockSpec((7, _C), lambda i: (0, 0)),
        ],
        out_specs=[
            pl.BlockSpec((_BLK, _HC), lambda i: (i, 0)),
            pl.BlockSpec((_BLK, _HC), lambda i: (i, 0)),
        ],
        out_shape=[jax.ShapeDtypeStruct((_NP, _HC), F32),
                   jax.ShapeDtypeStruct((_NP, _HC), F32)],
    )(xf, emb)


def _tc_stage(aggs, degi, dego, w0, w1, sums, save, first, resid, last):
    def body(*refs):
        it = iter(refs)
        aggf_lo, aggf_hi, aggb_lo, aggb_hi, degi_r, dego_r, w0_r, w1_r = (
            next(it) for _ in range(8))
        sum_r = sumsq_r = save_r = None
        if not first:
            sum_r, sumsq_r = next(it), next(it)
        if resid:
            save_r = next(it)
        if last:
            h_lo, h_hi = next(it), next(it)
        else:
            r_lo, r_hi, sum_o, sumsq_o = (next(it) for _ in range(4))
            save_o = next(it) if (first or resid) else None

        i = pl.program_id(0)
        aggf = jnp.concatenate([aggf_lo[...], aggf_hi[...]], axis=1)
        aggb = jnp.concatenate([aggb_lo[...], aggb_hi[...]], axis=1)
        norm = 1.0 / degi_r[:, 0:1]
        normt = 1.0 / dego_r[:, 0:1]
        w0v, w1v = w0_r[...], w1_r[...]
        if not first:
            m = jnp.sum(sum_r[...], axis=(0, 1)) * (1.0 / _N)
            var = jnp.sum(sumsq_r[...], axis=(0, 1)) * (1.0 / _N) - m * m
            inv = lax.rsqrt(var + _EPS)
            aggf = aggf * inv[None, :]
            aggb = aggb * inv[None, :]
        h = (jnp.dot(norm * aggf, w0v, precision=_PREC) +
             jnp.dot(normt * aggb, w1v, precision=_PREC))
        if not first:
            bias = jnp.dot((m * inv)[None, :], w0v + w1v, precision=_PREC)
            h = h - bias
        if resid:
            h = h + save_r[...]
        rowid = i * _BLK + lax.broadcasted_iota(jnp.int32, (_BLK, 1), 0)
        h = jnp.where(rowid < _N, h, 0.0)
        if last:
            h_lo[...] = h[:, :_HC]
            h_hi[...] = h[:, _HC:]
        else:
            if save_o is not None:
                save_o[...] = h
            r = jnp.maximum(h, 0.0)
            r_lo[...] = r[:, :_HC]
            r_hi[...] = r[:, _HC:]
            sum_o[...] = jnp.sum(r, axis=0, keepdims=True)[None]
            sumsq_o[...] = jnp.sum(r * r, axis=0, keepdims=True)[None]

    half = pl.BlockSpec((_BLK, _HC), lambda i: (i, 0))
    in_specs = [half, half, half, half,
                pl.BlockSpec((_BLK, 16), lambda i: (i, 0)),
                pl.BlockSpec((_BLK, 16), lambda i: (i, 0)),
                pl.BlockSpec((_C, _C), lambda i: (0, 0)),
                pl.BlockSpec((_C, _C), lambda i: (0, 0))]
    args = list(aggs) + [degi, dego, w0, w1]
    if not first:
        in_specs += [pl.BlockSpec((_NB, 1, _C), lambda i: (0, 0, 0))] * 2
        args += [sums[0], sums[1]]
    if resid:
        in_specs.append(pl.BlockSpec((_BLK, _C), lambda i: (i, 0)))
        args.append(save)
    if last:
        out_specs = [half, half]
        out_shape = [jax.ShapeDtypeStruct((_NP, _HC), F32)] * 2
    else:
        out_specs = [half, half,
                     pl.BlockSpec((1, 1, _C), lambda i: (i, 0, 0)),
                     pl.BlockSpec((1, 1, _C), lambda i: (i, 0, 0))]
        out_shape = [jax.ShapeDtypeStruct((_NP, _HC), F32),
                     jax.ShapeDtypeStruct((_NP, _HC), F32),
                     jax.ShapeDtypeStruct((_NB, 1, _C), F32),
                     jax.ShapeDtypeStruct((_NB, 1, _C), F32)]
        if first or resid:
            out_specs.append(pl.BlockSpec((_BLK, _C), lambda i: (i, 0)))
            out_shape.append(jax.ShapeDtypeStruct((_NP, _C), F32))
    return pl.pallas_call(
        body, grid=(_NB,), in_specs=in_specs, out_specs=out_specs,
        out_shape=out_shape,
    )(*args)


def _tc_head_body(p_lo, p_hi, cnt, f1w, f1b, f2w, f2b, f3w, f3b, out):
    p = jnp.concatenate([p_lo[...], p_hi[...]], axis=1) / cnt[...]
    g = jnp.maximum(jnp.dot(p, f1w[...], precision=_PREC) + f1b[...], 0.0)
    g = jnp.maximum(jnp.dot(g, f2w[...], precision=_PREC) + f2b[...], 0.0)
    out[...] = jnp.dot(g, f3w[...], precision=_PREC) + f3b[...]


def _tc_head(p_lo, p_hi, cnt, f1w, f1b, f2w, f2b, f3w, f3b):
    return pl.pallas_call(
        _tc_head_body,
        out_shape=jax.ShapeDtypeStruct((_G, 1), F32),
    )(p_lo, p_hi, cnt, f1w, f1b, f2w, f2b, f3w, f3b)


def kernel(x, sources, targets, batch, counts, total, emb, conv_w, res_w,
           fc1_w, fc1_b, fc2_w, fc2_b, fc3_w, fc3_b):
    del total
    pad = _NP - _N
    xf = jnp.pad(x, (0, pad)).astype(F32)[:, None]
    sources = sources.astype(jnp.int32)
    targets = targets.astype(jnp.int32)
    batchp = jnp.pad(batch.astype(jnp.int32), (0, pad))

    degi, dego = _sc_deg()(targets, sources)
    r_lo, r_hi = _tc_embed(xf, emb)

    stage_w = [(conv_w[0], conv_w[1])]
    for i in range(4):
        for j in range(2):
            stage_w.append((res_w[i, j, 0], res_w[i, j, 1]))

    save = None
    sums = None
    out_final = None
    for k in range(9):
        aggs = _sc_agg()(r_lo, r_hi, sources, targets)
        first = k == 0
        resid = k in (2, 4, 6, 8)
        last = k == 8
        w0, w1 = stage_w[k]
        outs = _tc_stage(aggs, degi, dego, w0, w1, sums, save,
                         first, resid, last)
        if last:
            out_final = outs
        else:
            r_lo, r_hi, s_p, ss_p = outs[:4]
            if first or resid:
                save = outs[4]
            sums = (s_p, ss_p)

    p_lo, p_hi = _sc_pool()(out_final[0], out_final[1], batchp)
    out2 = _tc_head(p_lo, p_hi, counts[:, None], fc1_w, fc1_b[None, :],
                    fc2_w, fc2_b[None, :], fc3_w, fc3_b[None, :])
    return jnp.squeeze(out2, axis=-1)
